# Initial kernel scaffold; baseline (speedup 1.0000x reference)
import jax
import jax.numpy as jnp
from jax.experimental import pallas as pl


def kernel(x, edge_index, batch, W_emb, b_emb, W1a, b1a, W2a, b2a, epsa, W1b, b1b, W2b, b2b, epsb, W1c, b1c, W2c, b2c, epsc, W_c1, b_c1, W_c2, b_c2):
    raise NotImplementedError("write your pallas kernel here")


# speedup vs baseline: 3.8166x; 3.8166x over previous
import functools

import jax
import jax.numpy as jnp
from jax import lax
from jax.experimental import pallas as pl
from jax.experimental.pallas import tpu as pltpu
from jax.experimental.pallas import tpu_sc as plsc

N = 50000
E = 800000
F_IN = 128
H = 64
HH = H // 2
C = 10
G = 512

BN = 512
NP = 50176
NBLK = NP // BN

NSUB = 16
NCORE = 2
CH = 128
SCK = 40
NSUP = 10
EPT = CH * SCK * NSUP
EP = EPT * NSUB
ACC_ROWS = 51200
ACC_PT = ACC_ROWS // NSUB
DUMMY = 50432

SEGR = 544
RPW = NP // NSUB
RCH = RPW // 2

_MESH = dict(core_axis_name="c", subcore_axis_name="s",
             num_cores=NCORE, num_subcores=NSUB)


def _emb_body(x_ref, w_ref, b_ref, out_ref):
    y = jnp.dot(x_ref[...], w_ref[...], preferred_element_type=jnp.float32)
    y = y + b_ref[...]
    out_ref[0] = y[:, :HH]
    out_ref[1] = y[:, HH:]


def _elu(y):
    return jnp.where(y > 0.0, y, jnp.exp(jnp.minimum(y, 0.0)) - 1.0)


def _mlp_body(hh_ref, agg_ref, w1_ref, b1_ref, w2_ref, b2_ref, eps_ref,
              out_ref):
    scale = 1.0 + eps_ref[0, 0]
    z0 = scale * hh_ref[0] + agg_ref[0]
    z1 = scale * hh_ref[1] + agg_ref[1]
    y1 = (jnp.dot(z0, w1_ref[:HH, :], preferred_element_type=jnp.float32)
          + jnp.dot(z1, w1_ref[HH:, :], preferred_element_type=jnp.float32)
          + b1_ref[...])
    y1 = jnp.maximum(y1, 0.0)
    y = jnp.dot(y1, w2_ref[...], preferred_element_type=jnp.float32)
    y = _elu(y + b2_ref[...])
    out_ref[0] = y[:, :HH]
    out_ref[1] = y[:, HH:]


def _cls_body(psum_ref, pmax_ref, pcnt_ref, w1_ref, b1_ref, w2_ref, b2_ref,
              out_ref):
    sums = jnp.sum(psum_ref[...], axis=1)
    maxs = jnp.max(pmax_ref[...], axis=1)
    cnts = jnp.sum(pcnt_ref[...], axis=1)
    cnt = cnts[0, :G, 0:1]
    ssum = jnp.concatenate([sums[0, :G, :], sums[1, :G, :]], axis=1)
    smax = jnp.concatenate([maxs[0, :G, :], maxs[1, :G, :]], axis=1)
    mean = ssum / jnp.maximum(cnt, 1.0)
    z = jnp.concatenate([mean, smax], axis=1)
    y = jnp.maximum(
        jnp.dot(z, w1_ref[...], preferred_element_type=jnp.float32)
        + b1_ref[...], 0.0)
    y = jnp.dot(y, w2_ref[...], preferred_element_type=jnp.float32)
    out_ref[...] = y + b2_ref[...]


def _agg_body(tbl, src2, dstr, zro, agg_out, idx_s, idx_d, rowbuf, acc, gsem):
    c = lax.axis_index("c")
    s = lax.axis_index("s")
    pltpu.sync_copy(zro, acc.at[pl.ds(s * ACC_PT, ACC_PT)])
    plsc.subcore_barrier()

    def superchunk(j, carry):
        pltpu.sync_copy(src2.at[c, s, pl.ds(j * SCK, SCK)], idx_s)
        pltpu.sync_copy(dstr.at[s, pl.ds(j * SCK, SCK)], idx_d)

        def chunk(k, carry2):
            pltpu.async_copy(tbl.at[idx_s.at[k]], rowbuf, gsem).wait()
            pltpu.sync_copy(rowbuf, acc.at[idx_d.at[k]], add=True)
            return carry2

        return lax.fori_loop(0, SCK, chunk, carry)

    lax.fori_loop(0, NSUP, superchunk, 0)
    plsc.subcore_barrier()
    base = s * ACC_PT
    tail = NP - (NSUB - 1) * ACC_PT
    pltpu.sync_copy(acc.at[pl.ds(base, tail)], agg_out.at[c, pl.ds(base, tail)])

    @pl.when(s < NSUB - 1)
    def _():
        pltpu.sync_copy(acc.at[pl.ds(base + tail, ACC_PT - tail)],
                        agg_out.at[c, pl.ds(base + tail, ACC_PT - tail)])


def _pool_body(tbl, batch, psum, pmax, pcnt, hbuf, bbuf, sumb, maxb, cntb):
    c = lax.axis_index("c")
    s = lax.axis_index("s")
    base = s * RPW

    zeros16 = jnp.zeros((16,), jnp.float32)
    neg16 = jnp.full((16,), -jnp.inf, jnp.float32)

    def init(r, carry):
        sumb[r, pl.ds(0, 16)] = zeros16
        sumb[r, pl.ds(16, 16)] = zeros16
        maxb[r, pl.ds(0, 16)] = neg16
        maxb[r, pl.ds(16, 16)] = neg16
        cntb[r] = zeros16
        return carry

    lax.fori_loop(0, SEGR, init, 0)

    pltpu.sync_copy(batch.at[pl.ds(base, RPW)], bbuf)

    ones16 = jnp.ones((16,), jnp.float32)

    def run_half(half):
        pltpu.sync_copy(tbl.at[pl.ds(c * NP + base + half * RCH, RCH)], hbuf)

        def group(t, carry):
            ids = bbuf[pl.ds(half * RCH + t * 16, 16)]
            for j in range(16):
                g = ids[j]
                i = t * 16 + j
                r0 = hbuf[i, pl.ds(0, 16)]
                r1 = hbuf[i, pl.ds(16, 16)]
                sumb[g, pl.ds(0, 16)] = sumb[g, pl.ds(0, 16)] + r0
                sumb[g, pl.ds(16, 16)] = sumb[g, pl.ds(16, 16)] + r1
                maxb[g, pl.ds(0, 16)] = jnp.maximum(maxb[g, pl.ds(0, 16)], r0)
                maxb[g, pl.ds(16, 16)] = jnp.maximum(maxb[g, pl.ds(16, 16)], r1)
                cntb[g] = cntb[g] + ones16
            return carry

        lax.fori_loop(0, RCH // 16, group, 0)

    run_half(0)
    run_half(1)

    pltpu.sync_copy(sumb, psum.at[c, s])
    pltpu.sync_copy(maxb, pmax.at[c, s])
    pltpu.sync_copy(cntb, pcnt.at[c, s])


def _make_sc_kernels():
    mesh = plsc.VectorSubcoreMesh(**_MESH)
    params = pltpu.CompilerParams(use_tc_tiling_on_sc=False)
    agg = functools.partial(
        pl.kernel, _agg_body, mesh=mesh, compiler_params=params,
        out_type=jax.ShapeDtypeStruct((NCORE, NP, HH), jnp.float32),
        scratch_types=[
            pltpu.VMEM((SCK, CH), jnp.int32),
            pltpu.VMEM((SCK, CH), jnp.int32),
            pltpu.VMEM((CH, HH), jnp.float32),
            pltpu.VMEM_SHARED((ACC_ROWS, HH), jnp.float32),
            pltpu.SemaphoreType.DMA,
        ],
    )()
    pool = functools.partial(
        pl.kernel, _pool_body, mesh=mesh, compiler_params=params,
        out_type=(
            jax.ShapeDtypeStruct((NCORE, NSUB, SEGR, HH), jnp.float32),
            jax.ShapeDtypeStruct((NCORE, NSUB, SEGR, HH), jnp.float32),
            jax.ShapeDtypeStruct((NCORE, NSUB, SEGR, 16), jnp.float32),
        ),
        scratch_types=[
            pltpu.VMEM((RCH, HH), jnp.float32),
            pltpu.VMEM((RPW,), jnp.int32),
            pltpu.VMEM((SEGR, HH), jnp.float32),
            pltpu.VMEM((SEGR, HH), jnp.float32),
            pltpu.VMEM((SEGR, 16), jnp.float32),
        ],
    )()
    return agg, pool


def kernel(x, edge_index, batch, W_emb, b_emb, W1a, b1a, W2a, b2a, epsa,
           W1b, b1b, W2b, b2b, epsb, W1c, b1c, W2c, b2c, epsc,
           W_c1, b_c1, W_c2, b_c2):
    f32 = jnp.float32

    x_p = jnp.pad(x, ((0, NP - N), (0, 0)))
    src = edge_index[0]
    dst = edge_index[1]
    src_p = jnp.concatenate([src, jnp.zeros((EP - E,), jnp.int32)])
    dst_p = jnp.concatenate([dst, jnp.full((EP - E,), DUMMY, jnp.int32)])
    src_r = src_p.reshape(NSUB, NSUP * SCK, CH)
    src2 = jnp.stack([src_r, src_r + NP])
    dst_r = dst_p.reshape(NSUB, NSUP * SCK, CH)
    batch_p = jnp.concatenate([batch, jnp.full((NP - N,), G, jnp.int32)])
    zro = jnp.zeros((ACC_PT, HH), f32)

    emb = pl.pallas_call(
        _emb_body,
        grid=(NBLK,),
        in_specs=[
            pl.BlockSpec((BN, F_IN), lambda i: (i, 0)),
            pl.BlockSpec((F_IN, H), lambda i: (0, 0)),
            pl.BlockSpec((1, H), lambda i: (0, 0)),
        ],
        out_specs=pl.BlockSpec((NCORE, BN, HH), lambda i: (0, i, 0)),
        out_shape=jax.ShapeDtypeStruct((NCORE, NP, HH), f32),
    )
    hh = emb(x_p, W_emb, b_emb.reshape(1, H))

    agg_k, pool_k = _make_sc_kernels()

    mlp = pl.pallas_call(
        _mlp_body,
        grid=(NBLK,),
        in_specs=[
            pl.BlockSpec((NCORE, BN, HH), lambda i: (0, i, 0)),
            pl.BlockSpec((NCORE, BN, HH), lambda i: (0, i, 0)),
            pl.BlockSpec((H, 2 * H), lambda i: (0, 0)),
            pl.BlockSpec((1, 2 * H), lambda i: (0, 0)),
            pl.BlockSpec((2 * H, H), lambda i: (0, 0)),
            pl.BlockSpec((1, H), lambda i: (0, 0)),
            pl.BlockSpec((1, 1), lambda i: (0, 0)),
        ],
        out_specs=pl.BlockSpec((NCORE, BN, HH), lambda i: (0, i, 0)),
        out_shape=jax.ShapeDtypeStruct((NCORE, NP, HH), f32),
    )

    for (W1, b1, W2, b2, eps) in ((W1a, b1a, W2a, b2a, epsa),
                                  (W1b, b1b, W2b, b2b, epsb),
                                  (W1c, b1c, W2c, b2c, epsc)):
        tbl = hh.reshape(NCORE * NP, HH)
        agg = agg_k(tbl, src2, dst_r, zro)
        hh = mlp(hh, agg, W1, b1.reshape(1, 2 * H), W2, b2.reshape(1, H),
                 eps.reshape(1, 1))

    tbl = hh.reshape(NCORE * NP, HH)
    psum, pmax, pcnt = pool_k(tbl, batch_p)

    cls = pl.pallas_call(
        _cls_body,
        out_shape=jax.ShapeDtypeStruct((G, C), f32),
    )
    return cls(psum, pmax, pcnt, W_c1, b_c1.reshape(1, H), W_c2,
               b_c2.reshape(1, C))


# speedup vs baseline: 5.0108x; 1.3129x over previous
import functools

import jax
import jax.numpy as jnp
from jax import lax
from jax.experimental import pallas as pl
from jax.experimental.pallas import tpu as pltpu
from jax.experimental.pallas import tpu_sc as plsc

N = 50000
E = 800000
F_IN = 128
H = 64
HH = H // 2
C = 10
G = 512

BN = 512
NP = 50176
NBLK = NP // BN

NSUB = 16
NCORE = 2
CH = 128
NCH = 400
SCK = 80
NSUP = NCH // SCK
EPT = CH * NCH
EP = EPT * NSUB
ACC_ROWS = 50304
ACC_PT = ACC_ROWS // NSUB
DUMMY = 50200

SEGR = 544
RPW = NP // NSUB
RCH = RPW // 2

_MESH = dict(core_axis_name="c", subcore_axis_name="s",
             num_cores=NCORE, num_subcores=NSUB)


def _emb_body(x_ref, w_ref, b_ref, out_ref):
    y = jnp.dot(x_ref[...], w_ref[...], preferred_element_type=jnp.float32)
    y = y + b_ref[...]
    out_ref[0] = y[:, :HH]
    out_ref[1] = y[:, HH:]


def _elu(y):
    return jnp.where(y > 0.0, y, jnp.exp(jnp.minimum(y, 0.0)) - 1.0)


def _mlp_body(hh_ref, agg_ref, w1_ref, b1_ref, w2_ref, b2_ref, eps_ref,
              out_ref):
    scale = 1.0 + eps_ref[0, 0]
    z0 = scale * hh_ref[0] + agg_ref[0]
    z1 = scale * hh_ref[1] + agg_ref[1]
    y1 = (jnp.dot(z0, w1_ref[:HH, :], preferred_element_type=jnp.float32)
          + jnp.dot(z1, w1_ref[HH:, :], preferred_element_type=jnp.float32)
          + b1_ref[...])
    y1 = jnp.maximum(y1, 0.0)
    y = jnp.dot(y1, w2_ref[...], preferred_element_type=jnp.float32)
    y = _elu(y + b2_ref[...])
    out_ref[0] = y[:, :HH]
    out_ref[1] = y[:, HH:]


def _cls_body(psum_ref, pmax_ref, pcnt_ref, w1_ref, b1_ref, w2_ref, b2_ref,
              out_ref):
    sums = jnp.sum(psum_ref[...], axis=1)
    maxs = jnp.max(pmax_ref[...], axis=1)
    cnts = jnp.sum(pcnt_ref[...], axis=1)
    cnt = cnts[0, :G, 0:1]
    ssum = jnp.concatenate([sums[0, :G, :], sums[1, :G, :]], axis=1)
    smax = jnp.concatenate([maxs[0, :G, :], maxs[1, :G, :]], axis=1)
    mean = ssum / jnp.maximum(cnt, 1.0)
    z = jnp.concatenate([mean, smax], axis=1)
    y = jnp.maximum(
        jnp.dot(z, w1_ref[...], preferred_element_type=jnp.float32)
        + b1_ref[...], 0.0)
    y = jnp.dot(y, w2_ref[...], preferred_element_type=jnp.float32)
    out_ref[...] = y + b2_ref[...]


def _agg_body(tbl, src2, dstr, zro, agg_out, idx_s, idx_d, rowbuf0, rowbuf1,
              acc, zsem, isem, gsem0, gsem1):
    c = lax.axis_index("c")
    s = lax.axis_index("s")
    pltpu.sync_copy(zro, acc.at[pl.ds(s * ACC_PT, ACC_PT)])
    plsc.subcore_barrier()

    def superchunk(j, carry):
        pltpu.async_copy(src2.at[c, s, pl.ds(j * SCK, SCK)], idx_s, isem)
        pltpu.async_copy(dstr.at[s, pl.ds(j * SCK, SCK)], idx_d, isem)
        pltpu.make_async_copy(src2.at[c, s, pl.ds(j * SCK, SCK)], idx_s,
                              isem).wait()
        pltpu.make_async_copy(dstr.at[s, pl.ds(j * SCK, SCK)], idx_d,
                              isem).wait()

        pltpu.async_copy(tbl.at[idx_s.at[0]], rowbuf0, gsem0)
        pltpu.async_copy(tbl.at[idx_s.at[1]], rowbuf1, gsem1)

        def pair(p, carry2):
            k0 = 2 * p
            k1 = k0 + 1
            pltpu.make_async_copy(tbl.at[idx_s.at[k0]], rowbuf0, gsem0).wait()
            pltpu.sync_copy(rowbuf0, acc.at[idx_d.at[k0]], add=True)

            @pl.when(k0 + 2 < SCK)
            def _():
                pltpu.async_copy(tbl.at[idx_s.at[k0 + 2]], rowbuf0, gsem0)

            pltpu.make_async_copy(tbl.at[idx_s.at[k1]], rowbuf1, gsem1).wait()
            pltpu.sync_copy(rowbuf1, acc.at[idx_d.at[k1]], add=True)

            @pl.when(k1 + 2 < SCK)
            def _():
                pltpu.async_copy(tbl.at[idx_s.at[k1 + 2]], rowbuf1, gsem1)

            return carry2

        lax.fori_loop(0, SCK // 2, pair, 0)
        return carry

    lax.fori_loop(0, NSUP, superchunk, 0)
    plsc.subcore_barrier()
    base = s * ACC_PT
    tail = NP - (NSUB - 1) * ACC_PT
    pltpu.sync_copy(acc.at[pl.ds(base, tail)], agg_out.at[c, pl.ds(base, tail)])

    @pl.when(s < NSUB - 1)
    def _():
        pltpu.sync_copy(acc.at[pl.ds(base + tail, ACC_PT - tail)],
                        agg_out.at[c, pl.ds(base + tail, ACC_PT - tail)])


def _pool_body(tbl, batch, psum, pmax, pcnt, hbuf, bbuf, sumb, maxb, cntb):
    c = lax.axis_index("c")
    s = lax.axis_index("s")
    base = s * RPW

    zeros16 = jnp.zeros((16,), jnp.float32)
    neg16 = jnp.full((16,), -jnp.inf, jnp.float32)

    def init(r, carry):
        sumb[r, pl.ds(0, 16)] = zeros16
        sumb[r, pl.ds(16, 16)] = zeros16
        maxb[r, pl.ds(0, 16)] = neg16
        maxb[r, pl.ds(16, 16)] = neg16
        cntb[r] = zeros16
        return carry

    lax.fori_loop(0, SEGR, init, 0)

    pltpu.sync_copy(batch.at[pl.ds(base, RPW)], bbuf)

    ones16 = jnp.ones((16,), jnp.float32)

    def run_half(half):
        pltpu.sync_copy(tbl.at[pl.ds(c * NP + base + half * RCH, RCH)], hbuf)

        def group(t, carry):
            ids = bbuf[pl.ds(half * RCH + t * 16, 16)]
            for j in range(16):
                g = ids[j]
                i = t * 16 + j
                r0 = hbuf[i, pl.ds(0, 16)]
                r1 = hbuf[i, pl.ds(16, 16)]
                sumb[g, pl.ds(0, 16)] = sumb[g, pl.ds(0, 16)] + r0
                sumb[g, pl.ds(16, 16)] = sumb[g, pl.ds(16, 16)] + r1
                maxb[g, pl.ds(0, 16)] = jnp.maximum(maxb[g, pl.ds(0, 16)], r0)
                maxb[g, pl.ds(16, 16)] = jnp.maximum(maxb[g, pl.ds(16, 16)], r1)
                cntb[g] = cntb[g] + ones16
            return carry

        lax.fori_loop(0, RCH // 16, group, 0)

    run_half(0)
    run_half(1)

    pltpu.sync_copy(sumb, psum.at[c, s])
    pltpu.sync_copy(maxb, pmax.at[c, s])
    pltpu.sync_copy(cntb, pcnt.at[c, s])


def _make_sc_kernels():
    mesh = plsc.VectorSubcoreMesh(**_MESH)
    params = pltpu.CompilerParams(use_tc_tiling_on_sc=False)
    agg = functools.partial(
        pl.kernel, _agg_body, mesh=mesh, compiler_params=params,
        out_type=jax.ShapeDtypeStruct((NCORE, NP, HH), jnp.float32),
        scratch_types=[
            pltpu.VMEM((SCK, CH), jnp.int32),
            pltpu.VMEM((SCK, CH), jnp.int32),
            pltpu.VMEM((CH, HH), jnp.float32),
            pltpu.VMEM((CH, HH), jnp.float32),
            pltpu.VMEM_SHARED((ACC_ROWS, HH), jnp.float32),
            pltpu.SemaphoreType.DMA,
            pltpu.SemaphoreType.DMA,
            pltpu.SemaphoreType.DMA,
            pltpu.SemaphoreType.DMA,
        ],
    )()
    pool = functools.partial(
        pl.kernel, _pool_body, mesh=mesh, compiler_params=params,
        out_type=(
            jax.ShapeDtypeStruct((NCORE, NSUB, SEGR, HH), jnp.float32),
            jax.ShapeDtypeStruct((NCORE, NSUB, SEGR, HH), jnp.float32),
            jax.ShapeDtypeStruct((NCORE, NSUB, SEGR, 16), jnp.float32),
        ),
        scratch_types=[
            pltpu.VMEM((RCH, HH), jnp.float32),
            pltpu.VMEM((RPW,), jnp.int32),
            pltpu.VMEM((SEGR, HH), jnp.float32),
            pltpu.VMEM((SEGR, HH), jnp.float32),
            pltpu.VMEM((SEGR, 16), jnp.float32),
        ],
    )()
    return agg, pool


def kernel(x, edge_index, batch, W_emb, b_emb, W1a, b1a, W2a, b2a, epsa,
           W1b, b1b, W2b, b2b, epsb, W1c, b1c, W2c, b2c, epsc,
           W_c1, b_c1, W_c2, b_c2):
    f32 = jnp.float32

    x_p = jnp.pad(x, ((0, NP - N), (0, 0)))
    src = edge_index[0]
    dst = edge_index[1]
    src_p = jnp.concatenate([src, jnp.zeros((EP - E,), jnp.int32)])
    dst_p = jnp.concatenate([dst, jnp.full((EP - E,), DUMMY, jnp.int32)])
    src_r = src_p.reshape(NSUB, NCH, CH)
    src2 = jnp.stack([src_r, src_r + NP])
    dst_r = dst_p.reshape(NSUB, NCH, CH)
    batch_p = jnp.concatenate([batch, jnp.full((NP - N,), G, jnp.int32)])
    zro = jnp.zeros((ACC_PT, HH), f32)

    emb = pl.pallas_call(
        _emb_body,
        grid=(NBLK,),
        in_specs=[
            pl.BlockSpec((BN, F_IN), lambda i: (i, 0)),
            pl.BlockSpec((F_IN, H), lambda i: (0, 0)),
            pl.BlockSpec((1, H), lambda i: (0, 0)),
        ],
        out_specs=pl.BlockSpec((NCORE, BN, HH), lambda i: (0, i, 0)),
        out_shape=jax.ShapeDtypeStruct((NCORE, NP, HH), f32),
    )
    hh = emb(x_p, W_emb, b_emb.reshape(1, H))

    agg_k, pool_k = _make_sc_kernels()

    mlp = pl.pallas_call(
        _mlp_body,
        grid=(NBLK,),
        in_specs=[
            pl.BlockSpec((NCORE, BN, HH), lambda i: (0, i, 0)),
            pl.BlockSpec((NCORE, BN, HH), lambda i: (0, i, 0)),
            pl.BlockSpec((H, 2 * H), lambda i: (0, 0)),
            pl.BlockSpec((1, 2 * H), lambda i: (0, 0)),
            pl.BlockSpec((2 * H, H), lambda i: (0, 0)),
            pl.BlockSpec((1, H), lambda i: (0, 0)),
            pl.BlockSpec((1, 1), lambda i: (0, 0)),
        ],
        out_specs=pl.BlockSpec((NCORE, BN, HH), lambda i: (0, i, 0)),
        out_shape=jax.ShapeDtypeStruct((NCORE, NP, HH), f32),
    )

    for (W1, b1, W2, b2, eps) in ((W1a, b1a, W2a, b2a, epsa),
                                  (W1b, b1b, W2b, b2b, epsb),
                                  (W1c, b1c, W2c, b2c, epsc)):
        tbl = hh.reshape(NCORE * NP, HH)
        agg = agg_k(tbl, src2, dst_r, zro)
        hh = mlp(hh, agg, W1, b1.reshape(1, 2 * H), W2, b2.reshape(1, H),
                 eps.reshape(1, 1))

    tbl = hh.reshape(NCORE * NP, HH)
    psum, pmax, pcnt = pool_k(tbl, batch_p)

    cls = pl.pallas_call(
        _cls_body,
        out_shape=jax.ShapeDtypeStruct((G, C), f32),
    )
    return cls(psum, pmax, pcnt, W_c1, b_c1.reshape(1, H), W_c2,
               b_c2.reshape(1, C))


# speedup vs baseline: 5.4412x; 1.0859x over previous
import functools

import jax
import jax.numpy as jnp
from jax import lax
from jax.experimental import pallas as pl
from jax.experimental.pallas import tpu as pltpu
from jax.experimental.pallas import tpu_sc as plsc

N = 50000
E = 800000
F_IN = 128
H = 64
HH = H // 2
C = 10
G = 512

BN = 1024
NP = 50176
NBLK = NP // BN

NSUB = 16
NCORE = 2
CH = 128
NCH = 400
SCK = 40
NSUP = NCH // SCK
NBUF = 4
EPT = CH * NCH
EP = EPT * NSUB
ACC_ROWS = 50304
ACC_PT = ACC_ROWS // NSUB
DUMMY = 50200

SEGR = 544
RPW = NP // NSUB
RCH = RPW // 2

_MESH = dict(core_axis_name="c", subcore_axis_name="s",
             num_cores=NCORE, num_subcores=NSUB)


def _emb_body(x_ref, w_ref, b_ref, out_ref):
    y = jnp.dot(x_ref[...], w_ref[...], preferred_element_type=jnp.float32)
    y = y + b_ref[...]
    out_ref[0] = y[:, :HH]
    out_ref[1] = y[:, HH:]


def _elu(y):
    return jnp.where(y > 0.0, y, jnp.exp(jnp.minimum(y, 0.0)) - 1.0)


def _mlp_body(hh_ref, agg_ref, w1_ref, b1_ref, w2_ref, b2_ref, eps_ref,
              out_ref):
    scale = 1.0 + eps_ref[0, 0]
    z0 = scale * hh_ref[0] + agg_ref[0]
    z1 = scale * hh_ref[1] + agg_ref[1]
    y1 = (jnp.dot(z0, w1_ref[:HH, :], preferred_element_type=jnp.float32)
          + jnp.dot(z1, w1_ref[HH:, :], preferred_element_type=jnp.float32)
          + b1_ref[...])
    y1 = jnp.maximum(y1, 0.0)
    y = jnp.dot(y1, w2_ref[...], preferred_element_type=jnp.float32)
    y = _elu(y + b2_ref[...])
    out_ref[0] = y[:, :HH]
    out_ref[1] = y[:, HH:]


def _cls_body(psum_ref, pmax_ref, pcnt_ref, w1_ref, b1_ref, w2_ref, b2_ref,
              out_ref):
    sums = jnp.sum(psum_ref[...], axis=1)
    maxs = jnp.max(pmax_ref[...], axis=1)
    cnts = jnp.sum(pcnt_ref[...], axis=1)
    cnt = cnts[0, :G, 0:1]
    ssum = jnp.concatenate([sums[0, :G, :], sums[1, :G, :]], axis=1)
    smax = jnp.concatenate([maxs[0, :G, :], maxs[1, :G, :]], axis=1)
    mean = ssum / jnp.maximum(cnt, 1.0)
    z = jnp.concatenate([mean, smax], axis=1)
    y = jnp.maximum(
        jnp.dot(z, w1_ref[...], preferred_element_type=jnp.float32)
        + b1_ref[...], 0.0)
    y = jnp.dot(y, w2_ref[...], preferred_element_type=jnp.float32)
    out_ref[...] = y + b2_ref[...]


def _agg_body(tbl, src2, dstr, zro, agg_out, idx_s, idx_d,
              rb0, rb1, rb2, rb3, acc, isem,
              gs0, gs1, gs2, gs3, ss0, ss1, ss2, ss3):
    c = lax.axis_index("c")
    s = lax.axis_index("s")
    rbufs = (rb0, rb1, rb2, rb3)
    gsems = (gs0, gs1, gs2, gs3)
    ssems = (ss0, ss1, ss2, ss3)
    pltpu.sync_copy(zro, acc.at[pl.ds(s * ACC_PT, ACC_PT)])
    plsc.subcore_barrier()

    def gather(k, b):
        pltpu.async_copy(tbl.at[idx_s.at[k]], rbufs[b], gsems[b])

    def gather_wait(b):
        pltpu.make_async_copy(tbl.at[idx_s.at[0]], rbufs[b], gsems[b]).wait()

    def scatter(k, b):
        pltpu.async_copy(rbufs[b], acc.at[idx_d.at[k]], ssems[b], add=True)

    def scatter_wait(b):
        pltpu.make_async_copy(rbufs[b], acc.at[idx_d.at[0]], ssems[b]).wait()

    def superchunk(j, carry):
        pltpu.async_copy(src2.at[c, s, pl.ds(j * SCK, SCK)], idx_s, isem)
        pltpu.async_copy(dstr.at[s, pl.ds(j * SCK, SCK)], idx_d, isem)
        pltpu.make_async_copy(src2.at[c, s, pl.ds(j * SCK, SCK)], idx_s,
                              isem).wait()
        pltpu.make_async_copy(dstr.at[s, pl.ds(j * SCK, SCK)], idx_d,
                              isem).wait()

        gather(0, 0)
        gather(1, 1)

        def quad(q, carry2):
            for b in range(NBUF):
                k = NBUF * q + b
                b2 = (b + 2) % NBUF
                gather_wait(b)
                scatter(k, b)

                @pl.when(k >= 2)
                def _():
                    scatter_wait(b2)

                @pl.when(k + 2 < SCK)
                def _():
                    gather(k + 2, b2)

            return carry2

        lax.fori_loop(0, SCK // NBUF, quad, 0)
        scatter_wait(2)
        scatter_wait(3)
        return carry

    lax.fori_loop(0, NSUP, superchunk, 0)
    plsc.subcore_barrier()
    base = s * ACC_PT
    tail = NP - (NSUB - 1) * ACC_PT
    pltpu.sync_copy(acc.at[pl.ds(base, tail)], agg_out.at[c, pl.ds(base, tail)])

    @pl.when(s < NSUB - 1)
    def _():
        pltpu.sync_copy(acc.at[pl.ds(base + tail, ACC_PT - tail)],
                        agg_out.at[c, pl.ds(base + tail, ACC_PT - tail)])


def _pool_body(tbl, batch, psum, pmax, pcnt, hbuf, bbuf, sumb, maxb, cntb):
    c = lax.axis_index("c")
    s = lax.axis_index("s")
    base = s * RPW

    zeros16 = jnp.zeros((16,), jnp.float32)
    neg16 = jnp.full((16,), -jnp.inf, jnp.float32)

    def init(r, carry):
        sumb[r, pl.ds(0, 16)] = zeros16
        sumb[r, pl.ds(16, 16)] = zeros16
        maxb[r, pl.ds(0, 16)] = neg16
        maxb[r, pl.ds(16, 16)] = neg16
        cntb[r] = zeros16
        return carry

    lax.fori_loop(0, SEGR, init, 0)

    pltpu.sync_copy(batch.at[pl.ds(base, RPW)], bbuf)

    ones16 = jnp.ones((16,), jnp.float32)

    def run_half(half):
        pltpu.sync_copy(tbl.at[pl.ds(c * NP + base + half * RCH, RCH)], hbuf)

        def group(t, carry):
            ids = bbuf[pl.ds(half * RCH + t * 16, 16)]
            for j in range(16):
                g = ids[j]
                i = t * 16 + j
                r0 = hbuf[i, pl.ds(0, 16)]
                r1 = hbuf[i, pl.ds(16, 16)]
                sumb[g, pl.ds(0, 16)] = sumb[g, pl.ds(0, 16)] + r0
                sumb[g, pl.ds(16, 16)] = sumb[g, pl.ds(16, 16)] + r1
                maxb[g, pl.ds(0, 16)] = jnp.maximum(maxb[g, pl.ds(0, 16)], r0)
                maxb[g, pl.ds(16, 16)] = jnp.maximum(maxb[g, pl.ds(16, 16)], r1)
                cntb[g] = cntb[g] + ones16
            return carry

        lax.fori_loop(0, RCH // 16, group, 0)

    run_half(0)
    run_half(1)

    pltpu.sync_copy(sumb, psum.at[c, s])
    pltpu.sync_copy(maxb, pmax.at[c, s])
    pltpu.sync_copy(cntb, pcnt.at[c, s])


def _make_sc_kernels():
    mesh = plsc.VectorSubcoreMesh(**_MESH)
    params = pltpu.CompilerParams(use_tc_tiling_on_sc=False)
    agg = functools.partial(
        pl.kernel, _agg_body, mesh=mesh, compiler_params=params,
        out_type=jax.ShapeDtypeStruct((NCORE, NP, HH), jnp.float32),
        scratch_types=(
            [pltpu.VMEM((SCK, CH), jnp.int32),
             pltpu.VMEM((SCK, CH), jnp.int32)]
            + [pltpu.VMEM((CH, HH), jnp.float32) for _ in range(NBUF)]
            + [pltpu.VMEM_SHARED((ACC_ROWS, HH), jnp.float32)]
            + [pltpu.SemaphoreType.DMA for _ in range(1 + 2 * NBUF)]
        ),
    )()
    pool = functools.partial(
        pl.kernel, _pool_body, mesh=mesh, compiler_params=params,
        out_type=(
            jax.ShapeDtypeStruct((NCORE, NSUB, SEGR, HH), jnp.float32),
            jax.ShapeDtypeStruct((NCORE, NSUB, SEGR, HH), jnp.float32),
            jax.ShapeDtypeStruct((NCORE, NSUB, SEGR, 16), jnp.float32),
        ),
        scratch_types=[
            pltpu.VMEM((RCH, HH), jnp.float32),
            pltpu.VMEM((RPW,), jnp.int32),
            pltpu.VMEM((SEGR, HH), jnp.float32),
            pltpu.VMEM((SEGR, HH), jnp.float32),
            pltpu.VMEM((SEGR, 16), jnp.float32),
        ],
    )()
    return agg, pool


def kernel(x, edge_index, batch, W_emb, b_emb, W1a, b1a, W2a, b2a, epsa,
           W1b, b1b, W2b, b2b, epsb, W1c, b1c, W2c, b2c, epsc,
           W_c1, b_c1, W_c2, b_c2):
    f32 = jnp.float32

    x_p = jnp.pad(x, ((0, NP - N), (0, 0)))
    src = edge_index[0]
    dst = edge_index[1]
    src_p = jnp.concatenate([src, jnp.zeros((EP - E,), jnp.int32)])
    dst_p = jnp.concatenate([dst, jnp.full((EP - E,), DUMMY, jnp.int32)])
    src_r = src_p.reshape(NSUB, NCH, CH)
    src2 = jnp.stack([src_r, src_r + NP])
    dst_r = dst_p.reshape(NSUB, NCH, CH)
    batch_p = jnp.concatenate([batch, jnp.full((NP - N,), G, jnp.int32)])
    zro = jnp.zeros((ACC_PT, HH), f32)

    emb = pl.pallas_call(
        _emb_body,
        grid=(NBLK,),
        in_specs=[
            pl.BlockSpec((BN, F_IN), lambda i: (i, 0)),
            pl.BlockSpec((F_IN, H), lambda i: (0, 0)),
            pl.BlockSpec((1, H), lambda i: (0, 0)),
        ],
        out_specs=pl.BlockSpec((NCORE, BN, HH), lambda i: (0, i, 0)),
        out_shape=jax.ShapeDtypeStruct((NCORE, NP, HH), f32),
    )
    hh = emb(x_p, W_emb, b_emb.reshape(1, H))

    agg_k, pool_k = _make_sc_kernels()

    mlp = pl.pallas_call(
        _mlp_body,
        grid=(NBLK,),
        in_specs=[
            pl.BlockSpec((NCORE, BN, HH), lambda i: (0, i, 0)),
            pl.BlockSpec((NCORE, BN, HH), lambda i: (0, i, 0)),
            pl.BlockSpec((H, 2 * H), lambda i: (0, 0)),
            pl.BlockSpec((1, 2 * H), lambda i: (0, 0)),
            pl.BlockSpec((2 * H, H), lambda i: (0, 0)),
            pl.BlockSpec((1, H), lambda i: (0, 0)),
            pl.BlockSpec((1, 1), lambda i: (0, 0)),
        ],
        out_specs=pl.BlockSpec((NCORE, BN, HH), lambda i: (0, i, 0)),
        out_shape=jax.ShapeDtypeStruct((NCORE, NP, HH), f32),
    )

    for (W1, b1, W2, b2, eps) in ((W1a, b1a, W2a, b2a, epsa),
                                  (W1b, b1b, W2b, b2b, epsb),
                                  (W1c, b1c, W2c, b2c, epsc)):
        tbl = hh.reshape(NCORE * NP, HH)
        agg = agg_k(tbl, src2, dst_r, zro)
        hh = mlp(hh, agg, W1, b1.reshape(1, 2 * H), W2, b2.reshape(1, H),
                 eps.reshape(1, 1))

    tbl = hh.reshape(NCORE * NP, HH)
    psum, pmax, pcnt = pool_k(tbl, batch_p)

    cls = pl.pallas_call(
        _cls_body,
        out_shape=jax.ShapeDtypeStruct((G, C), f32),
    )
    return cls(psum, pmax, pcnt, W_c1, b_c1.reshape(1, H), W_c2,
               b_c2.reshape(1, C))


# speedup vs baseline: 5.6622x; 1.0406x over previous
import functools

import jax
import jax.numpy as jnp
from jax import lax
from jax.experimental import pallas as pl
from jax.experimental.pallas import tpu as pltpu
from jax.experimental.pallas import tpu_sc as plsc

N = 50000
E = 800000
F_IN = 128
H = 64
HH = H // 2
C = 10
G = 512

BN = 1024
NP = 50176
NBLK = NP // BN

NSUB = 16
NCORE = 2
CH = 128
NCH = 400
SCK = 40
NSUP = NCH // SCK
NBUF = 4
EPT = CH * NCH
EP = EPT * NSUB
ACC_ROWS = 50304
ACC_PT = ACC_ROWS // NSUB
DUMMY = 50200

SEGR = 544
RPW = NP // NSUB
RCH = RPW // 2

_MESH = dict(core_axis_name="c", subcore_axis_name="s",
             num_cores=NCORE, num_subcores=NSUB)


def _emb_body(x_ref, w_ref, b_ref, out_ref):
    y = jnp.dot(x_ref[...], w_ref[...], preferred_element_type=jnp.float32)
    y = y + b_ref[...]
    out_ref[0] = y[:, :HH]
    out_ref[1] = y[:, HH:]


def _elu(y):
    return jnp.where(y > 0.0, y, jnp.exp(jnp.minimum(y, 0.0)) - 1.0)


def _mlp_body(hh_ref, agg_ref, w1_ref, b1_ref, w2_ref, b2_ref, eps_ref,
              out_ref):
    scale = 1.0 + eps_ref[0, 0]
    z0 = scale * hh_ref[0] + agg_ref[0]
    z1 = scale * hh_ref[1] + agg_ref[1]
    y1 = (jnp.dot(z0, w1_ref[:HH, :], preferred_element_type=jnp.float32)
          + jnp.dot(z1, w1_ref[HH:, :], preferred_element_type=jnp.float32)
          + b1_ref[...])
    y1 = jnp.maximum(y1, 0.0)
    y = jnp.dot(y1, w2_ref[...], preferred_element_type=jnp.float32)
    y = _elu(y + b2_ref[...])
    out_ref[0] = y[:, :HH]
    out_ref[1] = y[:, HH:]


def _cls_body(psum_ref, pmax_ref, pcnt_ref, w1_ref, b1_ref, w2_ref, b2_ref,
              out_ref):
    sums = jnp.sum(psum_ref[...], axis=1)
    maxs = jnp.max(pmax_ref[...], axis=1)
    cnts = jnp.sum(pcnt_ref[...], axis=1)
    cnt = cnts[0, :G, 0:1]
    ssum = jnp.concatenate([sums[0, :G, :], sums[1, :G, :]], axis=1)
    smax = jnp.concatenate([maxs[0, :G, :], maxs[1, :G, :]], axis=1)
    mean = ssum / jnp.maximum(cnt, 1.0)
    z = jnp.concatenate([mean, smax], axis=1)
    y = jnp.maximum(
        jnp.dot(z, w1_ref[...], preferred_element_type=jnp.float32)
        + b1_ref[...], 0.0)
    y = jnp.dot(y, w2_ref[...], preferred_element_type=jnp.float32)
    out_ref[...] = y + b2_ref[...]


def _agg_body(tbl, src2, dstr, zro, agg_out, idx_s, idx_d,
              rb0, rb1, rb2, rb3, acc, isem,
              gs0, gs1, gs2, gs3, ss0, ss1, ss2, ss3):
    c = lax.axis_index("c")
    s = lax.axis_index("s")
    rbufs = (rb0, rb1, rb2, rb3)
    gsems = (gs0, gs1, gs2, gs3)
    ssems = (ss0, ss1, ss2, ss3)
    pltpu.sync_copy(zro, acc.at[pl.ds(s * ACC_PT, ACC_PT)])
    plsc.subcore_barrier()

    def gather(k, b):
        pltpu.async_copy(tbl.at[idx_s.at[k]], rbufs[b], gsems[b])

    def gather_wait(b):
        pltpu.make_async_copy(tbl.at[idx_s.at[0]], rbufs[b], gsems[b]).wait()

    def scatter(k, b):
        pltpu.async_copy(rbufs[b], acc.at[idx_d.at[k]], ssems[b], add=True)

    def scatter_wait(b):
        pltpu.make_async_copy(rbufs[b], acc.at[idx_d.at[0]], ssems[b]).wait()

    def superchunk(j, carry):
        pltpu.async_copy(src2.at[c, s, pl.ds(j * SCK, SCK)], idx_s, isem)
        pltpu.async_copy(dstr.at[s, pl.ds(j * SCK, SCK)], idx_d, isem)
        pltpu.make_async_copy(src2.at[c, s, pl.ds(j * SCK, SCK)], idx_s,
                              isem).wait()
        pltpu.make_async_copy(dstr.at[s, pl.ds(j * SCK, SCK)], idx_d,
                              isem).wait()

        gather(0, 0)
        gather(1, 1)
        gather(2, 2)

        def quad(q, carry2):
            for b in range(NBUF):
                k = NBUF * q + b
                b3 = (b + 3) % NBUF
                gather_wait(b)
                scatter(k, b)

                @pl.when(k >= 1)
                def _():
                    scatter_wait(b3)

                @pl.when(k + 3 < SCK)
                def _():
                    gather(k + 3, b3)

            return carry2

        lax.fori_loop(0, SCK // NBUF, quad, 0)
        scatter_wait((SCK - 1) % NBUF)
        return carry

    lax.fori_loop(0, NSUP, superchunk, 0)
    plsc.subcore_barrier()
    base = s * ACC_PT
    tail = NP - (NSUB - 1) * ACC_PT
    pltpu.sync_copy(acc.at[pl.ds(base, tail)], agg_out.at[c, pl.ds(base, tail)])

    @pl.when(s < NSUB - 1)
    def _():
        pltpu.sync_copy(acc.at[pl.ds(base + tail, ACC_PT - tail)],
                        agg_out.at[c, pl.ds(base + tail, ACC_PT - tail)])


def _pool_body(tbl, batch, psum, pmax, pcnt, hbuf, bbuf, sumb, maxb, cntb):
    c = lax.axis_index("c")
    s = lax.axis_index("s")
    base = s * RPW

    zeros16 = jnp.zeros((16,), jnp.float32)
    neg16 = jnp.full((16,), -jnp.inf, jnp.float32)

    def init(r, carry):
        sumb[r, pl.ds(0, 16)] = zeros16
        sumb[r, pl.ds(16, 16)] = zeros16
        maxb[r, pl.ds(0, 16)] = neg16
        maxb[r, pl.ds(16, 16)] = neg16
        cntb[r] = zeros16
        return carry

    lax.fori_loop(0, SEGR, init, 0)

    pltpu.sync_copy(batch.at[pl.ds(base, RPW)], bbuf)

    ones16 = jnp.ones((16,), jnp.float32)

    def run_half(half):
        pltpu.sync_copy(tbl.at[pl.ds(c * NP + base + half * RCH, RCH)], hbuf)

        def group(t, carry):
            ids = bbuf[pl.ds(half * RCH + t * 16, 16)]
            for j in range(16):
                g = ids[j]
                i = t * 16 + j
                r0 = hbuf[i, pl.ds(0, 16)]
                r1 = hbuf[i, pl.ds(16, 16)]
                sumb[g, pl.ds(0, 16)] = sumb[g, pl.ds(0, 16)] + r0
                sumb[g, pl.ds(16, 16)] = sumb[g, pl.ds(16, 16)] + r1
                maxb[g, pl.ds(0, 16)] = jnp.maximum(maxb[g, pl.ds(0, 16)], r0)
                maxb[g, pl.ds(16, 16)] = jnp.maximum(maxb[g, pl.ds(16, 16)], r1)
                cntb[g] = cntb[g] + ones16
            return carry

        lax.fori_loop(0, RCH // 16, group, 0)

    run_half(0)
    run_half(1)

    pltpu.sync_copy(sumb, psum.at[c, s])
    pltpu.sync_copy(maxb, pmax.at[c, s])
    pltpu.sync_copy(cntb, pcnt.at[c, s])


def _make_sc_kernels():
    mesh = plsc.VectorSubcoreMesh(**_MESH)
    params = pltpu.CompilerParams(use_tc_tiling_on_sc=False)
    agg = functools.partial(
        pl.kernel, _agg_body, mesh=mesh, compiler_params=params,
        out_type=jax.ShapeDtypeStruct((NCORE, NP, HH), jnp.float32),
        scratch_types=(
            [pltpu.VMEM((SCK, CH), jnp.int32),
             pltpu.VMEM((SCK, CH), jnp.int32)]
            + [pltpu.VMEM((CH, HH), jnp.float32) for _ in range(NBUF)]
            + [pltpu.VMEM_SHARED((ACC_ROWS, HH), jnp.float32)]
            + [pltpu.SemaphoreType.DMA for _ in range(1 + 2 * NBUF)]
        ),
    )()
    pool = functools.partial(
        pl.kernel, _pool_body, mesh=mesh, compiler_params=params,
        out_type=(
            jax.ShapeDtypeStruct((NCORE, NSUB, SEGR, HH), jnp.float32),
            jax.ShapeDtypeStruct((NCORE, NSUB, SEGR, HH), jnp.float32),
            jax.ShapeDtypeStruct((NCORE, NSUB, SEGR, 16), jnp.float32),
        ),
        scratch_types=[
            pltpu.VMEM((RCH, HH), jnp.float32),
            pltpu.VMEM((RPW,), jnp.int32),
            pltpu.VMEM((SEGR, HH), jnp.float32),
            pltpu.VMEM((SEGR, HH), jnp.float32),
            pltpu.VMEM((SEGR, 16), jnp.float32),
        ],
    )()
    return agg, pool


def kernel(x, edge_index, batch, W_emb, b_emb, W1a, b1a, W2a, b2a, epsa,
           W1b, b1b, W2b, b2b, epsb, W1c, b1c, W2c, b2c, epsc,
           W_c1, b_c1, W_c2, b_c2):
    f32 = jnp.float32

    x_p = jnp.pad(x, ((0, NP - N), (0, 0)))
    src = edge_index[0]
    dst = edge_index[1]
    src_p = jnp.concatenate([src, jnp.zeros((EP - E,), jnp.int32)])
    dst_p = jnp.concatenate([dst, jnp.full((EP - E,), DUMMY, jnp.int32)])
    src_r = src_p.reshape(NSUB, NCH, CH)
    src2 = jnp.stack([src_r, src_r + NP])
    dst_r = dst_p.reshape(NSUB, NCH, CH)
    batch_p = jnp.concatenate([batch, jnp.full((NP - N,), G, jnp.int32)])
    zro = jnp.zeros((ACC_PT, HH), f32)

    emb = pl.pallas_call(
        _emb_body,
        grid=(NBLK,),
        in_specs=[
            pl.BlockSpec((BN, F_IN), lambda i: (i, 0)),
            pl.BlockSpec((F_IN, H), lambda i: (0, 0)),
            pl.BlockSpec((1, H), lambda i: (0, 0)),
        ],
        out_specs=pl.BlockSpec((NCORE, BN, HH), lambda i: (0, i, 0)),
        out_shape=jax.ShapeDtypeStruct((NCORE, NP, HH), f32),
    )
    hh = emb(x_p, W_emb, b_emb.reshape(1, H))

    agg_k, pool_k = _make_sc_kernels()

    mlp = pl.pallas_call(
        _mlp_body,
        grid=(NBLK,),
        in_specs=[
            pl.BlockSpec((NCORE, BN, HH), lambda i: (0, i, 0)),
            pl.BlockSpec((NCORE, BN, HH), lambda i: (0, i, 0)),
            pl.BlockSpec((H, 2 * H), lambda i: (0, 0)),
            pl.BlockSpec((1, 2 * H), lambda i: (0, 0)),
            pl.BlockSpec((2 * H, H), lambda i: (0, 0)),
            pl.BlockSpec((1, H), lambda i: (0, 0)),
            pl.BlockSpec((1, 1), lambda i: (0, 0)),
        ],
        out_specs=pl.BlockSpec((NCORE, BN, HH), lambda i: (0, i, 0)),
        out_shape=jax.ShapeDtypeStruct((NCORE, NP, HH), f32),
    )

    for (W1, b1, W2, b2, eps) in ((W1a, b1a, W2a, b2a, epsa),
                                  (W1b, b1b, W2b, b2b, epsb),
                                  (W1c, b1c, W2c, b2c, epsc)):
        tbl = hh.reshape(NCORE * NP, HH)
        agg = agg_k(tbl, src2, dst_r, zro)
        hh = mlp(hh, agg, W1, b1.reshape(1, 2 * H), W2, b2.reshape(1, H),
                 eps.reshape(1, 1))

    tbl = hh.reshape(NCORE * NP, HH)
    psum, pmax, pcnt = pool_k(tbl, batch_p)

    cls = pl.pallas_call(
        _cls_body,
        out_shape=jax.ShapeDtypeStruct((G, C), f32),
    )
    return cls(psum, pmax, pcnt, W_c1, b_c1.reshape(1, H), W_c2,
               b_c2.reshape(1, C))


# speedup vs baseline: 7.1955x; 1.2708x over previous
import functools

import jax
import jax.numpy as jnp
from jax import lax
from jax.experimental import pallas as pl
from jax.experimental.pallas import tpu as pltpu
from jax.experimental.pallas import tpu_sc as plsc

N = 50000
E = 800000
F_IN = 128
H = 64
HH = H // 2
C = 10
G = 512

BN = 1024
NP = 50176
NBLK = NP // BN

NSUB = 16
NCORE = 2
CH = 128
NCH = 400
SCK = 40
NSUP = NCH // SCK
NBUF = 4
EPT = CH * NCH
EP = EPT * NSUB
ACC_ROWS = 50304
ACC_PT = ACC_ROWS // NSUB
DUMMY = 50200

SEGR = 544
RPW = NP // NSUB
RCH = RPW // 2

_MESH = dict(core_axis_name="c", subcore_axis_name="s",
             num_cores=NCORE, num_subcores=NSUB)


def _emb_body(x_ref, w_ref, b_ref, out_ref, tb_ref):
    y = jnp.dot(x_ref[...], w_ref[...], preferred_element_type=jnp.float32)
    y = y + b_ref[...]
    out_ref[0] = y[:, :HH]
    out_ref[1] = y[:, HH:]
    tb_ref[0] = y[:, :HH].astype(jnp.bfloat16)
    tb_ref[1] = y[:, HH:].astype(jnp.bfloat16)


def _elu(y):
    return jnp.where(y > 0.0, y, jnp.exp(jnp.minimum(y, 0.0)) - 1.0)


def _mlp_body(hh_ref, agg_ref, w1_ref, b1_ref, w2_ref, b2_ref, eps_ref,
              out_ref, tb_ref):
    scale = 1.0 + eps_ref[0, 0]
    z0 = scale * hh_ref[0] + agg_ref[0].astype(jnp.float32)
    z1 = scale * hh_ref[1] + agg_ref[1].astype(jnp.float32)
    y1 = (jnp.dot(z0, w1_ref[:HH, :], preferred_element_type=jnp.float32)
          + jnp.dot(z1, w1_ref[HH:, :], preferred_element_type=jnp.float32)
          + b1_ref[...])
    y1 = jnp.maximum(y1, 0.0)
    y = jnp.dot(y1, w2_ref[...], preferred_element_type=jnp.float32)
    y = _elu(y + b2_ref[...])
    out_ref[0] = y[:, :HH]
    out_ref[1] = y[:, HH:]
    tb_ref[0] = y[:, :HH].astype(jnp.bfloat16)
    tb_ref[1] = y[:, HH:].astype(jnp.bfloat16)


def _cls_body(psum_ref, pmax_ref, pcnt_ref, w1_ref, b1_ref, w2_ref, b2_ref,
              out_ref):
    sums = jnp.sum(psum_ref[...], axis=1)
    maxs = jnp.max(pmax_ref[...], axis=1)
    cnts = jnp.sum(pcnt_ref[...], axis=1)
    cnt = cnts[0, :G, 0:1]
    ssum = jnp.concatenate([sums[0, :G, :], sums[1, :G, :]], axis=1)
    smax = jnp.concatenate([maxs[0, :G, :], maxs[1, :G, :]], axis=1)
    mean = ssum / jnp.maximum(cnt, 1.0)
    z = jnp.concatenate([mean, smax], axis=1)
    y = jnp.maximum(
        jnp.dot(z, w1_ref[...], preferred_element_type=jnp.float32)
        + b1_ref[...], 0.0)
    y = jnp.dot(y, w2_ref[...], preferred_element_type=jnp.float32)
    out_ref[...] = y + b2_ref[...]


def _agg_body(tbl, src2, dstr, zro, agg_out, idx_s, idx_d,
              rb0, rb1, rb2, rb3, acc, isem,
              gs0, gs1, gs2, gs3, ss0, ss1, ss2, ss3):
    c = lax.axis_index("c")
    s = lax.axis_index("s")
    rbufs = (rb0, rb1, rb2, rb3)
    gsems = (gs0, gs1, gs2, gs3)
    ssems = (ss0, ss1, ss2, ss3)
    pltpu.sync_copy(zro, acc.at[pl.ds(s * ACC_PT, ACC_PT)])
    plsc.subcore_barrier()

    def gather(k, b):
        pltpu.async_copy(tbl.at[idx_s.at[k]], rbufs[b], gsems[b])

    def gather_wait(b):
        pltpu.make_async_copy(tbl.at[idx_s.at[0]], rbufs[b], gsems[b]).wait()

    def scatter(k, b):
        pltpu.async_copy(rbufs[b], acc.at[idx_d.at[k]], ssems[b], add=True)

    def scatter_wait(b):
        pltpu.make_async_copy(rbufs[b], acc.at[idx_d.at[0]], ssems[b]).wait()

    def superchunk(j, carry):
        pltpu.async_copy(src2.at[c, s, pl.ds(j * SCK, SCK)], idx_s, isem)
        pltpu.async_copy(dstr.at[s, pl.ds(j * SCK, SCK)], idx_d, isem)
        pltpu.make_async_copy(src2.at[c, s, pl.ds(j * SCK, SCK)], idx_s,
                              isem).wait()
        pltpu.make_async_copy(dstr.at[s, pl.ds(j * SCK, SCK)], idx_d,
                              isem).wait()

        gather(0, 0)
        gather(1, 1)
        gather(2, 2)

        def quad(q, carry2):
            for b in range(NBUF):
                k = NBUF * q + b
                b3 = (b + 3) % NBUF
                gather_wait(b)
                scatter(k, b)

                @pl.when(k >= 1)
                def _():
                    scatter_wait(b3)

                @pl.when(k + 3 < SCK)
                def _():
                    gather(k + 3, b3)

            return carry2

        lax.fori_loop(0, SCK // NBUF, quad, 0)
        scatter_wait((SCK - 1) % NBUF)
        return carry

    lax.fori_loop(0, NSUP, superchunk, 0)
    plsc.subcore_barrier()
    base = s * ACC_PT
    tail = NP - (NSUB - 1) * ACC_PT
    pltpu.sync_copy(acc.at[pl.ds(base, tail)], agg_out.at[c, pl.ds(base, tail)])

    @pl.when(s < NSUB - 1)
    def _():
        pltpu.sync_copy(acc.at[pl.ds(base + tail, ACC_PT - tail)],
                        agg_out.at[c, pl.ds(base + tail, ACC_PT - tail)])


def _pool_body(tbl, batch, psum, pmax, pcnt, hbuf, bbuf, sumb, maxb, cntb):
    c = lax.axis_index("c")
    s = lax.axis_index("s")
    base = s * RPW

    zeros16 = jnp.zeros((16,), jnp.float32)
    neg16 = jnp.full((16,), -jnp.inf, jnp.float32)

    def init(r, carry):
        sumb[r, pl.ds(0, 16)] = zeros16
        sumb[r, pl.ds(16, 16)] = zeros16
        maxb[r, pl.ds(0, 16)] = neg16
        maxb[r, pl.ds(16, 16)] = neg16
        cntb[r] = zeros16
        return carry

    lax.fori_loop(0, SEGR, init, 0)

    pltpu.sync_copy(batch.at[pl.ds(base, RPW)], bbuf)

    ones16 = jnp.ones((16,), jnp.float32)

    def run_half(half):
        pltpu.sync_copy(tbl.at[pl.ds(c * NP + base + half * RCH, RCH)], hbuf)

        def group(t, carry):
            ids = bbuf[pl.ds(half * RCH + t * 16, 16)]
            for j in range(16):
                g = ids[j]
                i = t * 16 + j
                r0 = hbuf[i, pl.ds(0, 16)]
                r1 = hbuf[i, pl.ds(16, 16)]
                sumb[g, pl.ds(0, 16)] = sumb[g, pl.ds(0, 16)] + r0
                sumb[g, pl.ds(16, 16)] = sumb[g, pl.ds(16, 16)] + r1
                maxb[g, pl.ds(0, 16)] = jnp.maximum(maxb[g, pl.ds(0, 16)], r0)
                maxb[g, pl.ds(16, 16)] = jnp.maximum(maxb[g, pl.ds(16, 16)], r1)
                cntb[g] = cntb[g] + ones16
            return carry

        lax.fori_loop(0, RCH // 16, group, 0)

    run_half(0)
    run_half(1)

    pltpu.sync_copy(sumb, psum.at[c, s])
    pltpu.sync_copy(maxb, pmax.at[c, s])
    pltpu.sync_copy(cntb, pcnt.at[c, s])


def _make_sc_kernels():
    mesh = plsc.VectorSubcoreMesh(**_MESH)
    params = pltpu.CompilerParams(use_tc_tiling_on_sc=False)
    agg = functools.partial(
        pl.kernel, _agg_body, mesh=mesh, compiler_params=params,
        out_type=jax.ShapeDtypeStruct((NCORE, NP, HH), jnp.bfloat16),
        scratch_types=(
            [pltpu.VMEM((SCK, CH), jnp.int32),
             pltpu.VMEM((SCK, CH), jnp.int32)]
            + [pltpu.VMEM((CH, HH), jnp.bfloat16) for _ in range(NBUF)]
            + [pltpu.VMEM_SHARED((ACC_ROWS, HH), jnp.bfloat16)]
            + [pltpu.SemaphoreType.DMA for _ in range(1 + 2 * NBUF)]
        ),
    )()
    pool = functools.partial(
        pl.kernel, _pool_body, mesh=mesh, compiler_params=params,
        out_type=(
            jax.ShapeDtypeStruct((NCORE, NSUB, SEGR, HH), jnp.float32),
            jax.ShapeDtypeStruct((NCORE, NSUB, SEGR, HH), jnp.float32),
            jax.ShapeDtypeStruct((NCORE, NSUB, SEGR, 16), jnp.float32),
        ),
        scratch_types=[
            pltpu.VMEM((RCH, HH), jnp.float32),
            pltpu.VMEM((RPW,), jnp.int32),
            pltpu.VMEM((SEGR, HH), jnp.float32),
            pltpu.VMEM((SEGR, HH), jnp.float32),
            pltpu.VMEM((SEGR, 16), jnp.float32),
        ],
    )()
    return agg, pool


def kernel(x, edge_index, batch, W_emb, b_emb, W1a, b1a, W2a, b2a, epsa,
           W1b, b1b, W2b, b2b, epsb, W1c, b1c, W2c, b2c, epsc,
           W_c1, b_c1, W_c2, b_c2):
    f32 = jnp.float32

    x_p = jnp.pad(x, ((0, NP - N), (0, 0)))
    src = edge_index[0]
    dst = edge_index[1]
    src_p = jnp.concatenate([src, jnp.zeros((EP - E,), jnp.int32)])
    dst_p = jnp.concatenate([dst, jnp.full((EP - E,), DUMMY, jnp.int32)])
    src_r = src_p.reshape(NSUB, NCH, CH)
    src2 = jnp.stack([src_r, src_r + NP])
    dst_r = dst_p.reshape(NSUB, NCH, CH)
    batch_p = jnp.concatenate([batch, jnp.full((NP - N,), G, jnp.int32)])
    zro = jnp.zeros((ACC_PT, HH), jnp.bfloat16)

    emb = pl.pallas_call(
        _emb_body,
        grid=(NBLK,),
        in_specs=[
            pl.BlockSpec((BN, F_IN), lambda i: (i, 0)),
            pl.BlockSpec((F_IN, H), lambda i: (0, 0)),
            pl.BlockSpec((1, H), lambda i: (0, 0)),
        ],
        out_specs=(pl.BlockSpec((NCORE, BN, HH), lambda i: (0, i, 0)),
                   pl.BlockSpec((NCORE, BN, HH), lambda i: (0, i, 0))),
        out_shape=(jax.ShapeDtypeStruct((NCORE, NP, HH), f32),
                   jax.ShapeDtypeStruct((NCORE, NP, HH), jnp.bfloat16)),
    )
    hh, tblb = emb(x_p, W_emb, b_emb.reshape(1, H))

    agg_k, pool_k = _make_sc_kernels()

    mlp = pl.pallas_call(
        _mlp_body,
        grid=(NBLK,),
        in_specs=[
            pl.BlockSpec((NCORE, BN, HH), lambda i: (0, i, 0)),
            pl.BlockSpec((NCORE, BN, HH), lambda i: (0, i, 0)),
            pl.BlockSpec((H, 2 * H), lambda i: (0, 0)),
            pl.BlockSpec((1, 2 * H), lambda i: (0, 0)),
            pl.BlockSpec((2 * H, H), lambda i: (0, 0)),
            pl.BlockSpec((1, H), lambda i: (0, 0)),
            pl.BlockSpec((1, 1), lambda i: (0, 0)),
        ],
        out_specs=(pl.BlockSpec((NCORE, BN, HH), lambda i: (0, i, 0)),
                   pl.BlockSpec((NCORE, BN, HH), lambda i: (0, i, 0))),
        out_shape=(jax.ShapeDtypeStruct((NCORE, NP, HH), f32),
                   jax.ShapeDtypeStruct((NCORE, NP, HH), jnp.bfloat16)),
    )

    for (W1, b1, W2, b2, eps) in ((W1a, b1a, W2a, b2a, epsa),
                                  (W1b, b1b, W2b, b2b, epsb),
                                  (W1c, b1c, W2c, b2c, epsc)):
        tbl = tblb.reshape(NCORE * NP, HH)
        agg = agg_k(tbl, src2, dst_r, zro)
        hh, tblb = mlp(hh, agg, W1, b1.reshape(1, 2 * H), W2,
                       b2.reshape(1, H), eps.reshape(1, 1))

    tbl = hh.reshape(NCORE * NP, HH)
    psum, pmax, pcnt = pool_k(tbl, batch_p)

    cls = pl.pallas_call(
        _cls_body,
        out_shape=jax.ShapeDtypeStruct((G, C), f32),
    )
    return cls(psum, pmax, pcnt, W_c1, b_c1.reshape(1, H), W_c2,
               b_c2.reshape(1, C))
